# baseline (speedup 1.0000x reference)
import functools

import jax
import jax.numpy as jnp
from jax.experimental import pallas as pl
from jax.experimental.pallas import tpu as pltpu

_B = 16384
_C = 1000
_R2 = 1024
_C2 = 16000
_BR2 = 64


def _stream_kernel(x_ref, yl_ref, yp_ref):
    yp_ref[...] = x_ref[...] + yl_ref[...]


@functools.partial(jax.jit, static_argnums=())
def _run(output, y_labeled):
    xv = output.reshape(_R2, _C2)
    ylv = y_labeled.reshape(_R2, _C2)
    grid = (_R2 // _BR2,)
    ypv = pl.pallas_call(
        _stream_kernel,
        grid=grid,
        in_specs=[
            pl.BlockSpec((_BR2, _C2), lambda i: (i, 0)),
            pl.BlockSpec((_BR2, _C2), lambda i: (i, 0)),
        ],
        out_specs=pl.BlockSpec((_BR2, _C2), lambda i: (i, 0)),
        out_shape=jax.ShapeDtypeStruct((_R2, _C2), jnp.float32),
        compiler_params=pltpu.CompilerParams(
            dimension_semantics=("parallel",),
        ),
    )(xv, ylv)
    return jnp.float32(0.0), ypv.reshape(_B, _C)


def kernel(iteration, output, y_labeled):
    del iteration
    final_loss, y_pred = _run(output, y_labeled)
    return (final_loss, y_pred)


# speedup vs baseline: 5.6996x; 5.6996x over previous
import functools

import jax
import jax.numpy as jnp
from jax.experimental import pallas as pl
from jax.experimental.pallas import tpu as pltpu

_B = 16384
_C = 1000
_BL = 512


def _fused_kernel(x_ref, yl_ref, yp_ref, part_ref):
    x = x_ref[...]
    yl = yl_ref[...]
    m = jnp.max(x, axis=0, keepdims=True)
    e = jnp.exp(x - m)
    s = jnp.sum(e, axis=0, keepdims=True)
    yp_ref[...] = jnp.clip(e * (1.0 / s), 1e-4, 1.0 - 1e-4)
    lse = m + jnp.log(s)
    ce = lse * jnp.sum(yl, axis=0, keepdims=True) \
        - jnp.sum(yl * x, axis=0, keepdims=True)
    part_ref[0, 0, 0] = jnp.sum(ce)


@functools.partial(jax.jit, static_argnums=())
def _run(output, y_labeled):
    xt = output.T
    ylt = y_labeled.T
    grid = (_B // _BL,)
    yp_t, partials = pl.pallas_call(
        _fused_kernel,
        grid=grid,
        in_specs=[
            pl.BlockSpec((_C, _BL), lambda i: (0, i)),
            pl.BlockSpec((_C, _BL), lambda i: (0, i)),
        ],
        out_specs=[
            pl.BlockSpec((_C, _BL), lambda i: (0, i)),
            pl.BlockSpec((1, 1, 1), lambda i: (i, 0, 0), memory_space=pltpu.SMEM),
        ],
        out_shape=[
            jax.ShapeDtypeStruct((_C, _B), jnp.float32),
            jax.ShapeDtypeStruct((grid[0], 1, 1), jnp.float32),
        ],
        compiler_params=pltpu.CompilerParams(
            dimension_semantics=("parallel",),
        ),
    )(xt, ylt)
    return jnp.sum(partials) * (1.0 / _B), yp_t.T


def kernel(iteration, output, y_labeled):
    del iteration
    final_loss, y_pred = _run(output, y_labeled)
    return (final_loss, y_pred)


# speedup vs baseline: 5.8895x; 1.0333x over previous
import functools

import jax
import jax.numpy as jnp
from jax.experimental import pallas as pl
from jax.experimental.pallas import tpu as pltpu

_B = 16384
_C = 1000
_BL = 1024


def _fused_kernel(x_ref, yl_ref, yp_ref, part_ref):
    x = x_ref[...]
    yl = yl_ref[...]
    m = jnp.max(x, axis=0, keepdims=True)
    e = jnp.exp(x - m)
    s = jnp.sum(e, axis=0, keepdims=True)
    yp_ref[...] = jnp.clip(e * (1.0 / s), 1e-4, 1.0 - 1e-4)
    lse = m + jnp.log(s)
    ce = lse * jnp.sum(yl, axis=0, keepdims=True) \
        - jnp.sum(yl * x, axis=0, keepdims=True)
    part_ref[0, 0, 0] = jnp.sum(ce)


@functools.partial(jax.jit, static_argnums=())
def _run(output, y_labeled):
    xt = output.T
    ylt = y_labeled.T
    grid = (_B // _BL,)
    yp_t, partials = pl.pallas_call(
        _fused_kernel,
        grid=grid,
        in_specs=[
            pl.BlockSpec((_C, _BL), lambda i: (0, i)),
            pl.BlockSpec((_C, _BL), lambda i: (0, i)),
        ],
        out_specs=[
            pl.BlockSpec((_C, _BL), lambda i: (0, i)),
            pl.BlockSpec((1, 1, 1), lambda i: (i, 0, 0), memory_space=pltpu.SMEM),
        ],
        out_shape=[
            jax.ShapeDtypeStruct((_C, _B), jnp.float32),
            jax.ShapeDtypeStruct((grid[0], 1, 1), jnp.float32),
        ],
        compiler_params=pltpu.CompilerParams(
            dimension_semantics=("parallel",),
        ),
    )(xt, ylt)
    return jnp.sum(partials) * (1.0 / _B), yp_t.T


def kernel(iteration, output, y_labeled):
    del iteration
    final_loss, y_pred = _run(output, y_labeled)
    return (final_loss, y_pred)
